# baseline (speedup 1.0000x reference)
import functools

import jax
import jax.numpy as jnp
from jax import lax
from jax.experimental import pallas as pl
import jax.experimental.pallas.tpu as pltpu
from jax.experimental.pallas import tpu_sc as plsc

LAM = 0.5
NEG = -1e30
S_BLK = 2048
BISECT_ITERS = 30

SC_CORES = 2
SC_SUBCORES = 16
SC_WORKERS = SC_CORES * SC_SUBCORES
SC_LANES = 16
GATHER_G = 16


def _score_sparsegen_kernel(h_ref, pe_ref, wa_ref, st_ref, en_ref, ba_ref,
                            attn_ref, row_ref):
    i = pl.program_id(0)
    B = attn_ref.shape[0]
    S = attn_ref.shape[2]
    nblk = S // S_BLK
    last = B * nblk
    T = st_ref.shape[1]

    @pl.when(i < last)
    def _():
        b = i // nblk
        j = i % nblk
        h = h_ref[0]
        w1 = wa_ref[0, :]
        w2 = wa_ref[1, :]
        pe = pe_ref[0, 0, :]
        r32 = lambda x: x.astype(jnp.bfloat16).astype(jnp.float32)
        c = jnp.sum(r32(pe) * r32(w1))
        raw = jnp.sum(r32(h) * r32(w2)[None, :], axis=1, keepdims=True)
        pos = j * S_BLK + lax.broadcasted_iota(jnp.int32, (S_BLK, 1), 0)
        cov = jnp.zeros((S_BLK, 1), jnp.bool_)
        for t in range(T):
            cov = cov | ((pos >= st_ref[b, t]) & (pos < en_ref[b, t]))
        sc = jnp.where(cov, raw + (c + ba_ref[0]), NEG)
        row_ref[pl.ds(b, 1), pl.ds(j * S_BLK, S_BLK)] = jnp.transpose(sc, (1, 0))

    @pl.when(i == last)
    def _():
        _sparsegen_body(row_ref, st_ref, en_ref, attn_ref)


def _sparsegen_body(sc_ref, st_ref, en_ref, attn_ref):
    z = sc_ref[...]
    B, S = z.shape
    T = st_ref.shape[1]
    zmax = jnp.max(z, axis=1, keepdims=True)
    z = z - zmax
    pos = lax.broadcasted_iota(jnp.int32, (1, S), 1)
    rows = []
    for b in range(B):
        cb = jnp.zeros((1, S), jnp.float32)
        for t in range(T):
            cond = (pos >= st_ref[b, t]) & (pos < en_ref[b, t])
            cb = jnp.where(cond, 1.0, cb)
        rows.append(cb)
    m = jnp.concatenate(rows, axis=0)

    one_m_lam = jnp.float32(1.0 - LAM)
    lo0 = jnp.full((B, 1), -(1.0 - LAM), jnp.float32)
    hi0 = jnp.zeros((B, 1), jnp.float32)

    def body(_, carry):
        lo, hi = carry
        mid = 0.5 * (lo + hi)
        f = jnp.sum(jnp.maximum(z - mid, 0.0), axis=1, keepdims=True)
        pred = f > one_m_lam
        return jnp.where(pred, mid, lo), jnp.where(pred, hi, mid)

    lo, hi = lax.fori_loop(0, BISECT_ITERS, body, (lo0, hi0))
    t_est = 0.5 * (lo + hi)
    supp = z > t_est
    n = jnp.sum(supp.astype(jnp.float32), axis=1, keepdims=True)
    ssum = jnp.sum(jnp.where(supp, z, 0.0), axis=1, keepdims=True)
    tau = (ssum - one_m_lam) / n
    p = jnp.maximum(z - tau, 0.0) / one_m_lam
    attn_ref[:, 0, :] = p * m


def _sc_pool_kernel(attn_hbm, hid_hbm, out_hbm, attn_v, idx_v, w_v, rows_v,
                    acc_v, sem):
    wid = lax.axis_index("s") * SC_CORES + lax.axis_index("c")
    npos = attn_hbm.shape[0] // SC_WORKERS
    nvec = npos // SC_LANES
    base = wid * npos
    pltpu.sync_copy(attn_hbm.at[pl.ds(base, npos)], attn_v)

    zero16f = jnp.zeros((SC_LANES,), jnp.float32)
    zero16i = jnp.zeros((SC_LANES,), jnp.int32)
    UNROLL = 8

    @pl.loop(0, acc_v.shape[0] // (SC_LANES * UNROLL))
    def _(ci):
        for k in range(UNROLL):
            acc_v[pl.ds((ci * UNROLL + k) * SC_LANES, SC_LANES)] = zero16f

    lanes = lax.iota(jnp.int32, SC_LANES)

    def comp_body(vi, count):
        w = attn_v[pl.ds(vi * SC_LANES, SC_LANES)]
        msk = w > 0.0
        pos = (base + vi * SC_LANES) + lanes
        cum = plsc.cumsum(msk.astype(jnp.int32))
        dest = count + cum - 1
        plsc.store_scatter(idx_v, [dest], pos, mask=msk)
        plsc.store_scatter(w_v, [dest], w, mask=msk)
        return count + jnp.max(cum)

    count = lax.fori_loop(0, nvec, comp_body, jnp.int32(0))
    tail = count + lanes
    plsc.store_scatter(idx_v, [tail], jnp.zeros((SC_LANES,), jnp.int32))
    plsc.store_scatter(w_v, [tail], zero16f)
    nrounds = (count + GATHER_G - 1) // GATHER_G

    def round_body(r, carry):
        pltpu.async_copy(hid_hbm.at[idx_v.at[pl.ds(r * GATHER_G, GATHER_G)]],
                         rows_v, sem).wait()
        wvecs = [plsc.load_gather(
                     w_v, [jnp.full((SC_LANES,), r * GATHER_G + g, jnp.int32)])
                 for g in range(GATHER_G)]

        @pl.loop(0, acc_v.shape[0] // SC_LANES)
        def _(ci):
            s = pl.ds(ci * SC_LANES, SC_LANES)
            v = acc_v[s]
            for g in range(GATHER_G):
                v = v + wvecs[g] * rows_v[g, s]
            acc_v[s] = v

        return carry

    lax.fori_loop(0, nrounds, round_body, 0)
    pltpu.sync_copy(acc_v, out_hbm.at[wid])


def _proj_kernel(p_ref, w_ref, b_ref, o_ref):
    p = p_ref[...]
    B = o_ref.shape[0]
    per_b = p.shape[0] // B
    rows = [jnp.sum(p[b * per_b:(b + 1) * per_b], axis=0, keepdims=True)
            for b in range(B)]
    pooled = jnp.concatenate(rows, axis=0)
    acc = lax.dot_general(pooled.astype(jnp.bfloat16),
                          w_ref[...].astype(jnp.bfloat16),
                          (((1,), (0,)), ((), ())),
                          preferred_element_type=jnp.float32)
    o_ref[...] = jnp.tanh(acc + b_ref[...])


def kernel(hidden, token_idxs, pooled_entities, W_align, b_align, W_out, b_out):
    B, S, IN = hidden.shape
    OUT = W_out.shape[1]
    T = token_idxs.shape[2]
    nblk = S // S_BLK

    starts = token_idxs[0, :, :, 0].astype(jnp.int32)
    ends = token_idxs[0, :, :, 1].astype(jnp.int32)
    wa = W_align[:, 0].reshape(2, IN)
    pe3 = pooled_entities.reshape(B, 1, IN)
    ba = b_align.astype(jnp.float32)

    last = B * nblk

    def _h_map(i):
        ic = jnp.minimum(i, last - 1)
        return (ic // nblk, ic % nblk, 0)

    def _pe_map(i):
        ic = jnp.minimum(i, last - 1)
        return (ic // nblk, 0, 0)

    attn = pl.pallas_call(
        _score_sparsegen_kernel,
        grid=(last + 1,),
        in_specs=[
            pl.BlockSpec((1, S_BLK, IN), _h_map),
            pl.BlockSpec((1, 1, IN), _pe_map),
            pl.BlockSpec((2, IN), lambda i: (0, 0)),
            pl.BlockSpec(memory_space=pltpu.SMEM),
            pl.BlockSpec(memory_space=pltpu.SMEM),
            pl.BlockSpec(memory_space=pltpu.SMEM),
        ],
        out_specs=pl.BlockSpec((B, 1, S), lambda i: (0, 0, 0)),
        out_shape=jax.ShapeDtypeStruct((B, 1, S), jnp.float32),
        scratch_shapes=[pltpu.VMEM((B, S), jnp.float32)],
    )(hidden, pe3, wa, starts, ends, ba)

    npos = (B * S) // SC_WORKERS
    sc_mesh = plsc.VectorSubcoreMesh(core_axis_name="c", subcore_axis_name="s")
    partial = pl.kernel(
        _sc_pool_kernel,
        out_type=jax.ShapeDtypeStruct((SC_WORKERS, IN), jnp.float32),
        mesh=sc_mesh,
        compiler_params=pltpu.CompilerParams(needs_layout_passes=False),
        scratch_types=[
            pltpu.VMEM((npos,), jnp.float32),
            pltpu.VMEM((npos + SC_LANES,), jnp.int32),
            pltpu.VMEM((npos + SC_LANES,), jnp.float32),
            pltpu.VMEM((GATHER_G, IN), jnp.float32),
            pltpu.VMEM((IN,), jnp.float32),
            pltpu.SemaphoreType.DMA,
        ],
    )(attn.reshape(B * S), hidden.reshape(B * S, IN))

    projected = pl.pallas_call(
        _proj_kernel,
        in_specs=[
            pl.BlockSpec(memory_space=pltpu.VMEM),
            pl.BlockSpec(memory_space=pltpu.VMEM),
            pl.BlockSpec(memory_space=pltpu.VMEM),
        ],
        out_specs=pl.BlockSpec(memory_space=pltpu.VMEM),
        out_shape=jax.ShapeDtypeStruct((B, OUT), jnp.float32),
    )(partial, W_out, b_out.reshape(1, OUT))

    return projected, attn.reshape(B, S, 1)


# speedup vs baseline: 1.0614x; 1.0614x over previous
import functools

import jax
import jax.numpy as jnp
from jax import lax
from jax.experimental import pallas as pl
import jax.experimental.pallas.tpu as pltpu
from jax.experimental.pallas import tpu_sc as plsc

LAM = 0.5
NEG = -1e30
S_BLK = 2048
BISECT_ITERS = 30

SC_CORES = 2
SC_SUBCORES = 16
SC_WORKERS = SC_CORES * SC_SUBCORES
SC_LANES = 16
GATHER_G = 8


def _score_sparsegen_kernel(h_ref, pe_ref, wa_ref, st_ref, en_ref, ba_ref,
                            attn_ref, row_ref):
    i = pl.program_id(0)
    B = attn_ref.shape[0]
    S = attn_ref.shape[2]
    nblk = S // S_BLK
    last = B * nblk
    T = st_ref.shape[1]

    @pl.when(i < last)
    def _():
        b = i // nblk
        j = i % nblk
        h = h_ref[0]
        w1 = wa_ref[0, :]
        w2 = wa_ref[1, :]
        pe = pe_ref[0, 0, :]
        r32 = lambda x: x.astype(jnp.bfloat16).astype(jnp.float32)
        c = jnp.sum(r32(pe) * r32(w1))
        raw = jnp.sum(r32(h) * r32(w2)[None, :], axis=1, keepdims=True)
        pos = j * S_BLK + lax.broadcasted_iota(jnp.int32, (S_BLK, 1), 0)
        cov = jnp.zeros((S_BLK, 1), jnp.bool_)
        for t in range(T):
            cov = cov | ((pos >= st_ref[b, t]) & (pos < en_ref[b, t]))
        sc = jnp.where(cov, raw + (c + ba_ref[0]), NEG)
        row_ref[pl.ds(b, 1), pl.ds(j * S_BLK, S_BLK)] = jnp.transpose(sc, (1, 0))

    @pl.when(i == last)
    def _():
        _sparsegen_body(row_ref, st_ref, en_ref, attn_ref)


def _sparsegen_body(sc_ref, st_ref, en_ref, attn_ref):
    z = sc_ref[...]
    B, S = z.shape
    T = st_ref.shape[1]
    zmax = jnp.max(z, axis=1, keepdims=True)
    z = z - zmax
    pos = lax.broadcasted_iota(jnp.int32, (1, S), 1)
    rows = []
    for b in range(B):
        cb = jnp.zeros((1, S), jnp.float32)
        for t in range(T):
            cond = (pos >= st_ref[b, t]) & (pos < en_ref[b, t])
            cb = jnp.where(cond, 1.0, cb)
        rows.append(cb)
    m = jnp.concatenate(rows, axis=0)

    one_m_lam = jnp.float32(1.0 - LAM)
    lo0 = jnp.full((B, 1), -(1.0 - LAM), jnp.float32)
    hi0 = jnp.zeros((B, 1), jnp.float32)

    def body(_, carry):
        lo, hi = carry
        mid = 0.5 * (lo + hi)
        f = jnp.sum(jnp.maximum(z - mid, 0.0), axis=1, keepdims=True)
        pred = f > one_m_lam
        return jnp.where(pred, mid, lo), jnp.where(pred, hi, mid)

    lo, hi = lax.fori_loop(0, BISECT_ITERS, body, (lo0, hi0))
    t_est = 0.5 * (lo + hi)
    supp = z > t_est
    n = jnp.sum(supp.astype(jnp.float32), axis=1, keepdims=True)
    ssum = jnp.sum(jnp.where(supp, z, 0.0), axis=1, keepdims=True)
    tau = (ssum - one_m_lam) / n
    p = jnp.maximum(z - tau, 0.0) / one_m_lam
    attn_ref[:, 0, :] = p * m


def _sc_pool_kernel(attn_hbm, hid_hbm, out_hbm, attn_v, idx_v, w_v, rows_v,
                    acc_v, sem):
    wid = lax.axis_index("s") * SC_CORES + lax.axis_index("c")
    npos = attn_hbm.shape[0] // SC_WORKERS
    nvec = npos // SC_LANES
    base = wid * npos
    pltpu.sync_copy(attn_hbm.at[pl.ds(base, npos)], attn_v)

    zero16f = jnp.zeros((SC_LANES,), jnp.float32)
    zero16i = jnp.zeros((SC_LANES,), jnp.int32)
    UNROLL = 8

    @pl.loop(0, acc_v.shape[0] // (SC_LANES * UNROLL))
    def _(ci):
        for k in range(UNROLL):
            acc_v[pl.ds((ci * UNROLL + k) * SC_LANES, SC_LANES)] = zero16f

    lanes = lax.iota(jnp.int32, SC_LANES)

    def comp_body(vi, count):
        w = attn_v[pl.ds(vi * SC_LANES, SC_LANES)]
        msk = w > 0.0
        pos = (base + vi * SC_LANES) + lanes
        cum = plsc.cumsum(msk.astype(jnp.int32))
        dest = count + cum - 1
        plsc.store_scatter(idx_v, [dest], pos, mask=msk)
        plsc.store_scatter(w_v, [dest], w, mask=msk)
        return count + jnp.max(cum)

    count = lax.fori_loop(0, nvec, comp_body, jnp.int32(0))
    tail = count + lanes
    plsc.store_scatter(idx_v, [tail], jnp.zeros((SC_LANES,), jnp.int32))
    plsc.store_scatter(w_v, [tail], zero16f)
    nrounds = (count + GATHER_G - 1) // GATHER_G

    def round_body(r, carry):
        pltpu.async_copy(hid_hbm.at[idx_v.at[pl.ds(r * GATHER_G, GATHER_G)]],
                         rows_v, sem).wait()
        wvecs = [plsc.load_gather(
                     w_v, [jnp.full((SC_LANES,), r * GATHER_G + g, jnp.int32)])
                 for g in range(GATHER_G)]

        @pl.loop(0, acc_v.shape[0] // SC_LANES)
        def _(ci):
            s = pl.ds(ci * SC_LANES, SC_LANES)
            v = acc_v[s]
            for g in range(GATHER_G):
                v = v + wvecs[g] * rows_v[g, s]
            acc_v[s] = v

        return carry

    lax.fori_loop(0, nrounds, round_body, 0)
    pltpu.sync_copy(acc_v, out_hbm.at[wid])


def _proj_kernel(p_ref, w_ref, b_ref, o_ref):
    p = p_ref[...]
    B = o_ref.shape[0]
    per_b = p.shape[0] // B
    rows = [jnp.sum(p[b * per_b:(b + 1) * per_b], axis=0, keepdims=True)
            for b in range(B)]
    pooled = jnp.concatenate(rows, axis=0)
    acc = lax.dot_general(pooled.astype(jnp.bfloat16),
                          w_ref[...].astype(jnp.bfloat16),
                          (((1,), (0,)), ((), ())),
                          preferred_element_type=jnp.float32)
    o_ref[...] = jnp.tanh(acc + b_ref[...])


def kernel(hidden, token_idxs, pooled_entities, W_align, b_align, W_out, b_out):
    B, S, IN = hidden.shape
    OUT = W_out.shape[1]
    T = token_idxs.shape[2]
    nblk = S // S_BLK

    starts = token_idxs[0, :, :, 0].astype(jnp.int32)
    ends = token_idxs[0, :, :, 1].astype(jnp.int32)
    wa = W_align[:, 0].reshape(2, IN)
    pe3 = pooled_entities.reshape(B, 1, IN)
    ba = b_align.astype(jnp.float32)

    last = B * nblk

    def _h_map(i):
        ic = jnp.minimum(i, last - 1)
        return (ic // nblk, ic % nblk, 0)

    def _pe_map(i):
        ic = jnp.minimum(i, last - 1)
        return (ic // nblk, 0, 0)

    attn = pl.pallas_call(
        _score_sparsegen_kernel,
        grid=(last + 1,),
        in_specs=[
            pl.BlockSpec((1, S_BLK, IN), _h_map),
            pl.BlockSpec((1, 1, IN), _pe_map),
            pl.BlockSpec((2, IN), lambda i: (0, 0)),
            pl.BlockSpec(memory_space=pltpu.SMEM),
            pl.BlockSpec(memory_space=pltpu.SMEM),
            pl.BlockSpec(memory_space=pltpu.SMEM),
        ],
        out_specs=pl.BlockSpec((B, 1, S), lambda i: (0, 0, 0)),
        out_shape=jax.ShapeDtypeStruct((B, 1, S), jnp.float32),
        scratch_shapes=[pltpu.VMEM((B, S), jnp.float32)],
    )(hidden, pe3, wa, starts, ends, ba)

    npos = (B * S) // SC_WORKERS
    sc_mesh = plsc.VectorSubcoreMesh(core_axis_name="c", subcore_axis_name="s")
    partial = pl.kernel(
        _sc_pool_kernel,
        out_type=jax.ShapeDtypeStruct((SC_WORKERS, IN), jnp.float32),
        mesh=sc_mesh,
        compiler_params=pltpu.CompilerParams(needs_layout_passes=False),
        scratch_types=[
            pltpu.VMEM((npos,), jnp.float32),
            pltpu.VMEM((npos + SC_LANES,), jnp.int32),
            pltpu.VMEM((npos + SC_LANES,), jnp.float32),
            pltpu.VMEM((GATHER_G, IN), jnp.float32),
            pltpu.VMEM((IN,), jnp.float32),
            pltpu.SemaphoreType.DMA,
        ],
    )(attn.reshape(B * S), hidden.reshape(B * S, IN))

    projected = pl.pallas_call(
        _proj_kernel,
        in_specs=[
            pl.BlockSpec(memory_space=pltpu.VMEM),
            pl.BlockSpec(memory_space=pltpu.VMEM),
            pl.BlockSpec(memory_space=pltpu.VMEM),
        ],
        out_specs=pl.BlockSpec(memory_space=pltpu.VMEM),
        out_shape=jax.ShapeDtypeStruct((B, OUT), jnp.float32),
    )(partial, W_out, b_out.reshape(1, OUT))

    return projected, attn.reshape(B, S, 1)
